# baseline (speedup 1.0000x reference)
import jax
import jax.numpy as jnp
from jax import lax
from jax.experimental import pallas as pl
from jax.experimental.pallas import tpu as pltpu
from jax.experimental.pallas import tpu_sc as plsc

N = 10000
E = 320000
EMB = 128
N_CLASSES = 12
N_GRAPHS = 64

NC = 2
NS = 16
NW = NC * NS
EPT = E // NW
CHUNK = 64
NSPL = 4
HC = CHUNK // NSPL
NCHUNK = 157
PADE = NCHUNK * CHUNK - EPT
S_ROWS = 624
NBUF = 2
NPBUF = 6
PLEAD = 4
BLK = 6
MAIN = 150
NBLK = MAIN // BLK

_f32 = jnp.float32


def _mlp_block(z, W1, b1, g1, be1, W2, b2, g2, be2):
    y = jnp.dot(z, W1, preferred_element_type=jnp.float32) + b1
    m = jnp.mean(y, axis=0, keepdims=True)
    v = jnp.mean((y - m) ** 2, axis=0, keepdims=True)
    h1 = jnp.maximum(g1 * (y - m) / jnp.sqrt(v + 1e-5) + be1, 0.0)
    y2 = jnp.dot(h1, W2, preferred_element_type=jnp.float32) + b2
    m2 = jnp.mean(y2, axis=0, keepdims=True)
    v2 = jnp.mean((y2 - m2) ** 2, axis=0, keepdims=True)
    return jnp.maximum(g2 * (y2 - m2) / jnp.sqrt(v2 + 1e-5) + be2, 0.0)


def _pool_from(h2, WoT, bo, batch_row):
    yT = lax.dot_general(WoT, h2, (((1,), (1,)), ((), ())),
                         preferred_element_type=jnp.float32) + bo
    gids = lax.broadcasted_iota(jnp.int32, (N_GRAPHS, 1), 0)
    mask = batch_row == gids
    cols = []
    for c in range(N_CLASSES):
        vc = jnp.where(mask, yT[c:c + 1, :], float("-inf"))
        cols.append(jnp.max(vc, axis=1, keepdims=True))
    return jnp.concatenate(cols, axis=1)


def _init_body(x_ref, W1, b1, g1, be1, W2, b2, g2, be2, WoT, bo, batch_ref,
               h_ref, out_ref):
    h2 = _mlp_block(x_ref[...], W1[...], b1[...], g1[...], be1[...],
                    W2[...], b2[...], g2[...], be2[...])
    h_ref[...] = h2
    out_ref[...] = _pool_from(h2, WoT[...], bo[...], batch_ref[...])


def _stage_body(h_ref, aggs_ref, sc_ref, W1, b1, g1, be1, W2, b2, g2, be2,
                WoT, bo, batch_ref, outprev_ref, h_new_ref, out_new_ref):
    z = h_ref[...] * sc_ref[...] + aggs_ref[0:N, :] + aggs_ref[N:2 * N, :]
    h2 = _mlp_block(z, W1[...], b1[...], g1[...], be1[...],
                    W2[...], b2[...], g2[...], be2[...])
    h_new_ref[...] = h2
    out_new_ref[...] = outprev_ref[...] + _pool_from(
        h2, WoT[...], bo[...], batch_ref[...])


def _tc_init(x, w, batch_row):
    return pl.pallas_call(
        _init_body,
        out_shape=(jax.ShapeDtypeStruct((N, EMB), _f32),
                   jax.ShapeDtypeStruct((N_GRAPHS, N_CLASSES), _f32)),
    )(x, *w, batch_row)


def _tc_stage(h, aggs, scale, w, batch_row, out_prev):
    return pl.pallas_call(
        _stage_body,
        out_shape=(jax.ShapeDtypeStruct((N, EMB), _f32),
                   jax.ShapeDtypeStruct((N_GRAPHS, N_CLASSES), _f32)),
    )(h, aggs, scale, *w, batch_row, out_prev)


def _sc_agg_body(h_hbm, pk_hbm, w_hbm, z_hbm, out_hbm, *refs):
    pbuf = refs[0:NPBUF]
    wbuf = refs[NPBUF:2 * NPBUF]
    rin = refs[2 * NPBUF:2 * NPBUF + NBUF]
    rout = refs[2 * NPBUF + NBUF:2 * NPBUF + 2 * NBUF]
    aggsh = refs[2 * NPBUF + 2 * NBUF]
    _sems = refs[2 * NPBUF + 2 * NBUF + 1:]
    isem = _sems[0:NPBUF]
    gsem = _sems[NPBUF:NPBUF + NSPL * NBUF]
    ssem = _sems[NPBUF + NSPL * NBUF:NPBUF + 2 * NSPL * NBUF]

    c = lax.axis_index("c")
    s = lax.axis_index("s")
    wid = c * NS + s

    pltpu.sync_copy(z_hbm.at[pl.ds(0, S_ROWS)],
                    aggsh.at[pl.ds(s * S_ROWS, S_ROWS)])

    @pl.when(s == 0)
    def _zero_tail():
        pltpu.sync_copy(z_hbm.at[pl.ds(0, N - NS * S_ROWS)],
                        aggsh.at[pl.ds(NS * S_ROWS, N - NS * S_ROWS)])

    plsc.subcore_barrier()

    def p_start(ci, bp):
        pltpu.async_copy(pk_hbm.at[wid * NCHUNK + ci], pbuf[bp], isem[bp])
        pltpu.async_copy(w_hbm.at[wid * NCHUNK + ci], wbuf[bp], isem[bp])

    def p_wait(ci, bp):
        pltpu.make_async_copy(pk_hbm.at[wid * NCHUNK + ci], pbuf[bp],
                              isem[bp]).wait()
        pltpu.make_async_copy(w_hbm.at[wid * NCHUNK + ci], wbuf[bp],
                              isem[bp]).wait()

    def g_start(bp, b2, hf):
        pltpu.async_copy(h_hbm.at[pbuf[bp].at[hf]],
                         rin[b2].at[pl.ds(hf * HC, HC)],
                         gsem[NSPL * b2 + hf])

    def g_wait(bp, b2, hf):
        pltpu.make_async_copy(h_hbm.at[pbuf[bp].at[hf]],
                              rin[b2].at[pl.ds(hf * HC, HC)],
                              gsem[NSPL * b2 + hf]).wait()

    def s_start(bp, b2, hf):
        pltpu.async_copy(rout[b2].at[pl.ds(hf * HC, HC)],
                         aggsh.at[pbuf[bp].at[NSPL + hf]],
                         ssem[NSPL * b2 + hf], add=True)

    def s_wait(bp, b2, hf):
        pltpu.make_async_copy(rout[b2].at[pl.ds(hf * HC, HC)],
                              aggsh.at[pbuf[bp].at[NSPL + hf]],
                              ssem[NSPL * b2 + hf]).wait()

    def mul(bp, b2, hf):
        def grp(m, cc):
            w16 = wbuf[bp][pl.ds(hf * HC + m * 16, 16)]
            for e in range(16):
                wgt = w16[e]
                r = hf * HC + m * 16 + e
                for j in range(EMB // 16):
                    rout[b2][r, pl.ds(j * 16, 16)] = (
                        rin[b2][r, pl.ds(j * 16, 16)] * wgt)
            return cc

        lax.fori_loop(0, HC // 16, grp, 0)

    def step(ci, b, drain, pstart, gstart):
        b2 = b % NBUF
        dbp = (b - NBUF) % NPBUF
        for hf in range(NSPL):
            g_wait(b, b2, hf)
            if drain is not False:
                if drain is True:
                    s_wait(dbp, b2, hf)
                else:
                    @pl.when(drain)
                    def _drain():
                        s_wait(dbp, b2, hf)
            mul(b, b2, hf)
            s_start(b, b2, hf)
        if pstart:
            p_start(ci + PLEAD, (b + PLEAD) % NPBUF)
        if gstart:
            p_wait(ci + NBUF, (b + NBUF) % NPBUF)
            for hf in range(NSPL):
                g_start((b + NBUF) % NPBUF, (b + NBUF) % NBUF, hf)

    for i in range(PLEAD):
        p_start(i, i)
    for i in range(NBUF):
        p_wait(i, i)
        for hf in range(NSPL):
            g_start(i, i, hf)

    def block(g, carry):
        for b in range(BLK):
            drain = True if b >= NBUF else (g > 0)
            step(g * BLK + b, b, drain, True, True)
        return carry

    lax.fori_loop(0, NBLK, block, 0)

    for t in range(MAIN, NCHUNK):
        step(t, t % BLK, True, t + PLEAD < NCHUNK, t + NBUF < NCHUNK)
    for t in range(NCHUNK - NBUF, NCHUNK):
        for hf in range(NSPL):
            s_wait(t % NPBUF, t % NBUF, hf)

    plsc.subcore_barrier()
    pltpu.sync_copy(aggsh.at[pl.ds(s * S_ROWS, S_ROWS)],
                    out_hbm.at[pl.ds(c * N + s * S_ROWS, S_ROWS)])

    @pl.when(s == 0)
    def _copy_tail():
        pltpu.sync_copy(aggsh.at[pl.ds(NS * S_ROWS, N - NS * S_ROWS)],
                        out_hbm.at[pl.ds(c * N + NS * S_ROWS, N - NS * S_ROWS)])


def _sc_agg(h, packed, wts, zrs):
    mesh = plsc.VectorSubcoreMesh(core_axis_name="c", subcore_axis_name="s")
    k = pl.kernel(
        _sc_agg_body, mesh=mesh,
        out_type=jax.ShapeDtypeStruct((2 * N, EMB), _f32),
        scratch_types=(
            [pltpu.VMEM((2 * NSPL, HC), jnp.int32)] * NPBUF
            + [pltpu.VMEM((CHUNK,), _f32)] * NPBUF
            + [pltpu.VMEM((CHUNK, EMB), _f32)] * (2 * NBUF)
            + [pltpu.VMEM_SHARED((N, EMB), _f32)]
            + [pltpu.SemaphoreType.DMA] * (NPBUF + 2 * NSPL * NBUF)
        ),
    )
    return k(h, packed, wts, zrs)


def _stage_weights(mlp, out_lin):
    return (mlp["lin1"]["W"], mlp["lin1"]["b"].reshape(1, EMB),
            mlp["bn1"]["gamma"].reshape(1, EMB), mlp["bn1"]["beta"].reshape(1, EMB),
            mlp["lin2"]["W"], mlp["lin2"]["b"].reshape(1, EMB),
            mlp["bn2"]["gamma"].reshape(1, EMB), mlp["bn2"]["beta"].reshape(1, EMB),
            out_lin["W"].T, out_lin["b"].reshape(N_CLASSES, 1))


def kernel(x, edge_index, edge_attr, edge_weight, batch, params):
    del edge_attr
    pad2 = ((0, 0), (0, PADE))
    src = jnp.pad(edge_index[0].astype(jnp.int32).reshape(NW, EPT), pad2)
    dst = jnp.pad(edge_index[1].astype(jnp.int32).reshape(NW, EPT), pad2)
    packed = jnp.concatenate(
        [src.reshape(NW * NCHUNK, NSPL, HC),
         dst.reshape(NW * NCHUNK, NSPL, HC)], axis=1)
    wts = jnp.pad(edge_weight.astype(_f32).reshape(NW, EPT),
                  pad2).reshape(NW * NCHUNK, CHUNK)
    zrs = jnp.zeros((S_ROWS, EMB), _f32)
    batch_row = batch.astype(jnp.int32).reshape(1, N)

    h, out = _tc_init(
        x, _stage_weights(params["init_mlp"], params["init_lin"]), batch_row)
    for lp in params["layers"]:
        aggs = _sc_agg(h, packed, wts, zrs)
        scale = (1.0 + lp["eps"]).astype(_f32).reshape(1, 1)
        h, out = _tc_stage(h, aggs, scale,
                           _stage_weights(lp["mlp"], lp["out_lin"]),
                           batch_row, out)
    return out


# speedup vs baseline: 1.2472x; 1.2472x over previous
import jax
import jax.numpy as jnp
from jax import lax
from jax.experimental import pallas as pl
from jax.experimental.pallas import tpu as pltpu
from jax.experimental.pallas import tpu_sc as plsc

N = 10000
E = 320000
EMB = 128
N_CLASSES = 12
N_GRAPHS = 64

NC = 2
NS = 16
NW = NC * NS
EPT = E // NW
CHUNK = 64
NSPL = 2
HC = CHUNK // NSPL
NCHUNK = 157
PADE = NCHUNK * CHUNK - EPT
S_ROWS = 624
NBUF = 2
NPBUF = 6
PLEAD = 4
BLK = 6
MAIN = 150
NBLK = MAIN // BLK

_f32 = jnp.float32


def _mlp_block(z, W1, b1, g1, be1, W2, b2, g2, be2):
    y = jnp.dot(z, W1, preferred_element_type=jnp.float32) + b1
    m = jnp.mean(y, axis=0, keepdims=True)
    v = jnp.mean((y - m) ** 2, axis=0, keepdims=True)
    h1 = jnp.maximum(g1 * (y - m) / jnp.sqrt(v + 1e-5) + be1, 0.0)
    y2 = jnp.dot(h1, W2, preferred_element_type=jnp.float32) + b2
    m2 = jnp.mean(y2, axis=0, keepdims=True)
    v2 = jnp.mean((y2 - m2) ** 2, axis=0, keepdims=True)
    return jnp.maximum(g2 * (y2 - m2) / jnp.sqrt(v2 + 1e-5) + be2, 0.0)


def _pool_from(h2, WoT, bo, batch_row):
    yT = lax.dot_general(WoT, h2, (((1,), (1,)), ((), ())),
                         preferred_element_type=jnp.float32) + bo
    gids = lax.broadcasted_iota(jnp.int32, (N_GRAPHS, 1), 0)
    mask = batch_row == gids
    cols = []
    for c in range(N_CLASSES):
        vc = jnp.where(mask, yT[c:c + 1, :], float("-inf"))
        cols.append(jnp.max(vc, axis=1, keepdims=True))
    return jnp.concatenate(cols, axis=1)


def _init_body(x_ref, W1, b1, g1, be1, W2, b2, g2, be2, WoT, bo, batch_ref,
               h_ref, out_ref):
    h2 = _mlp_block(x_ref[...], W1[...], b1[...], g1[...], be1[...],
                    W2[...], b2[...], g2[...], be2[...])
    h_ref[...] = h2
    out_ref[...] = _pool_from(h2, WoT[...], bo[...], batch_ref[...])


def _stage_body(h_ref, aggs_ref, sc_ref, W1, b1, g1, be1, W2, b2, g2, be2,
                WoT, bo, batch_ref, outprev_ref, h_new_ref, out_new_ref):
    z = h_ref[...] * sc_ref[...] + aggs_ref[0:N, :] + aggs_ref[N:2 * N, :]
    h2 = _mlp_block(z, W1[...], b1[...], g1[...], be1[...],
                    W2[...], b2[...], g2[...], be2[...])
    h_new_ref[...] = h2
    out_new_ref[...] = outprev_ref[...] + _pool_from(
        h2, WoT[...], bo[...], batch_ref[...])


def _tc_init(x, w, batch_row):
    return pl.pallas_call(
        _init_body,
        out_shape=(jax.ShapeDtypeStruct((N, EMB), _f32),
                   jax.ShapeDtypeStruct((N_GRAPHS, N_CLASSES), _f32)),
    )(x, *w, batch_row)


def _tc_stage(h, aggs, scale, w, batch_row, out_prev):
    return pl.pallas_call(
        _stage_body,
        out_shape=(jax.ShapeDtypeStruct((N, EMB), _f32),
                   jax.ShapeDtypeStruct((N_GRAPHS, N_CLASSES), _f32)),
    )(h, aggs, scale, *w, batch_row, out_prev)


def _sc_agg_body(h_hbm, pk_hbm, w_hbm, z_hbm, out_hbm, *refs):
    pbuf = refs[0:NPBUF]
    wbuf = refs[NPBUF:2 * NPBUF]
    rin = refs[2 * NPBUF:2 * NPBUF + NBUF]
    rout = refs[2 * NPBUF + NBUF:2 * NPBUF + 2 * NBUF]
    aggsh = refs[2 * NPBUF + 2 * NBUF]
    _sems = refs[2 * NPBUF + 2 * NBUF + 1:]
    isem = _sems[0:NPBUF]
    gsem = _sems[NPBUF:NPBUF + NSPL * NBUF]
    ssem = _sems[NPBUF + NSPL * NBUF:NPBUF + 2 * NSPL * NBUF]

    c = lax.axis_index("c")
    s = lax.axis_index("s")
    wid = c * NS + s

    pltpu.sync_copy(z_hbm.at[pl.ds(0, S_ROWS)],
                    aggsh.at[pl.ds(s * S_ROWS, S_ROWS)])

    @pl.when(s == 0)
    def _zero_tail():
        pltpu.sync_copy(z_hbm.at[pl.ds(0, N - NS * S_ROWS)],
                        aggsh.at[pl.ds(NS * S_ROWS, N - NS * S_ROWS)])

    plsc.subcore_barrier()

    def p_start(ci, bp):
        pltpu.async_copy(pk_hbm.at[wid * NCHUNK + ci], pbuf[bp], isem[bp])
        pltpu.async_copy(w_hbm.at[wid * NCHUNK + ci], wbuf[bp], isem[bp])

    def p_wait(ci, bp):
        pltpu.make_async_copy(pk_hbm.at[wid * NCHUNK + ci], pbuf[bp],
                              isem[bp]).wait()
        pltpu.make_async_copy(w_hbm.at[wid * NCHUNK + ci], wbuf[bp],
                              isem[bp]).wait()

    def g_start(bp, b2, hf):
        pltpu.async_copy(h_hbm.at[pbuf[bp].at[hf]],
                         rin[b2].at[pl.ds(hf * HC, HC)],
                         gsem[NSPL * b2 + hf])

    def g_wait(bp, b2, hf):
        pltpu.make_async_copy(h_hbm.at[pbuf[bp].at[hf]],
                              rin[b2].at[pl.ds(hf * HC, HC)],
                              gsem[NSPL * b2 + hf]).wait()

    def s_start(bp, b2, hf):
        pltpu.async_copy(rout[b2].at[pl.ds(hf * HC, HC)],
                         aggsh.at[pbuf[bp].at[NSPL + hf]],
                         ssem[NSPL * b2 + hf], add=True)

    def s_wait(bp, b2, hf):
        pltpu.make_async_copy(rout[b2].at[pl.ds(hf * HC, HC)],
                              aggsh.at[pbuf[bp].at[NSPL + hf]],
                              ssem[NSPL * b2 + hf]).wait()

    def mul(bp, b2, hf):
        def grp(m, cc):
            w16 = wbuf[bp][pl.ds(hf * HC + m * 16, 16)]
            for e in range(16):
                wgt = w16[e]
                r = hf * HC + m * 16 + e
                for j in range(EMB // 16):
                    rout[b2][r, pl.ds(j * 16, 16)] = (
                        rin[b2][r, pl.ds(j * 16, 16)] * wgt)
            return cc

        lax.fori_loop(0, HC // 16, grp, 0)

    def step(ci, b, drain, pstart, gstart):
        b2 = b % NBUF
        dbp = (b - NBUF) % NPBUF
        for hf in range(NSPL):
            g_wait(b, b2, hf)
            if drain is not False:
                if drain is True:
                    s_wait(dbp, b2, hf)
                else:
                    @pl.when(drain)
                    def _drain():
                        s_wait(dbp, b2, hf)
            mul(b, b2, hf)
            s_start(b, b2, hf)
            if gstart:
                if hf == 0:
                    p_wait(ci + NBUF, (b + NBUF) % NPBUF)
                g_start((b + NBUF) % NPBUF, (b + NBUF) % NBUF, hf)
        if pstart:
            p_start(ci + PLEAD, (b + PLEAD) % NPBUF)

    for i in range(PLEAD):
        p_start(i, i)
    for i in range(NBUF):
        p_wait(i, i)
        for hf in range(NSPL):
            g_start(i, i, hf)

    def block(g, carry):
        for b in range(BLK):
            drain = True if b >= NBUF else (g > 0)
            step(g * BLK + b, b, drain, True, True)
        return carry

    lax.fori_loop(0, NBLK, block, 0)

    for t in range(MAIN, NCHUNK):
        step(t, t % BLK, True, t + PLEAD < NCHUNK, t + NBUF < NCHUNK)
    for t in range(NCHUNK - NBUF, NCHUNK):
        for hf in range(NSPL):
            s_wait(t % NPBUF, t % NBUF, hf)

    plsc.subcore_barrier()
    pltpu.sync_copy(aggsh.at[pl.ds(s * S_ROWS, S_ROWS)],
                    out_hbm.at[pl.ds(c * N + s * S_ROWS, S_ROWS)])

    @pl.when(s == 0)
    def _copy_tail():
        pltpu.sync_copy(aggsh.at[pl.ds(NS * S_ROWS, N - NS * S_ROWS)],
                        out_hbm.at[pl.ds(c * N + NS * S_ROWS, N - NS * S_ROWS)])


def _sc_agg(h, packed, wts, zrs):
    mesh = plsc.VectorSubcoreMesh(core_axis_name="c", subcore_axis_name="s")
    k = pl.kernel(
        _sc_agg_body, mesh=mesh,
        out_type=jax.ShapeDtypeStruct((2 * N, EMB), _f32),
        scratch_types=(
            [pltpu.VMEM((2 * NSPL, HC), jnp.int32)] * NPBUF
            + [pltpu.VMEM((CHUNK,), _f32)] * NPBUF
            + [pltpu.VMEM((CHUNK, EMB), _f32)] * (2 * NBUF)
            + [pltpu.VMEM_SHARED((N, EMB), _f32)]
            + [pltpu.SemaphoreType.DMA] * (NPBUF + 2 * NSPL * NBUF)
        ),
    )
    return k(h, packed, wts, zrs)


def _stage_weights(mlp, out_lin):
    return (mlp["lin1"]["W"], mlp["lin1"]["b"].reshape(1, EMB),
            mlp["bn1"]["gamma"].reshape(1, EMB), mlp["bn1"]["beta"].reshape(1, EMB),
            mlp["lin2"]["W"], mlp["lin2"]["b"].reshape(1, EMB),
            mlp["bn2"]["gamma"].reshape(1, EMB), mlp["bn2"]["beta"].reshape(1, EMB),
            out_lin["W"].T, out_lin["b"].reshape(N_CLASSES, 1))


def kernel(x, edge_index, edge_attr, edge_weight, batch, params):
    del edge_attr
    pad2 = ((0, 0), (0, PADE))
    src = jnp.pad(edge_index[0].astype(jnp.int32).reshape(NW, EPT), pad2)
    dst = jnp.pad(edge_index[1].astype(jnp.int32).reshape(NW, EPT), pad2)
    packed = jnp.concatenate(
        [src.reshape(NW * NCHUNK, NSPL, HC),
         dst.reshape(NW * NCHUNK, NSPL, HC)], axis=1)
    wts = jnp.pad(edge_weight.astype(_f32).reshape(NW, EPT),
                  pad2).reshape(NW * NCHUNK, CHUNK)
    zrs = jnp.zeros((S_ROWS, EMB), _f32)
    batch_row = batch.astype(jnp.int32).reshape(1, N)

    h, out = _tc_init(
        x, _stage_weights(params["init_mlp"], params["init_lin"]), batch_row)
    for lp in params["layers"]:
        aggs = _sc_agg(h, packed, wts, zrs)
        scale = (1.0 + lp["eps"]).astype(_f32).reshape(1, 1)
        h, out = _tc_stage(h, aggs, scale,
                           _stage_weights(lp["mlp"], lp["out_lin"]),
                           batch_row, out)
    return out
